# Initial kernel scaffold; baseline (speedup 1.0000x reference)
import jax
import jax.numpy as jnp
from jax.experimental import pallas as pl


def kernel(hit_score, hit_embedding, hit_particle_id):
    raise NotImplementedError("write your pallas kernel here")


# speedup vs baseline: 1.2647x; 1.2647x over previous
import jax
import jax.numpy as jnp
from jax import lax
from jax.experimental import pallas as pl
from jax.experimental.pallas import tpu as pltpu

_QMIN = 0.01
_SB = 0.1
_N = 20000
_D = 8
_T = 1536
_B = 400
_NB = _N // _B
_BIG = 1 << 30


def _body(x_ref, beta_ref, oid_ref,
          o_loss, o_va, o_vr, o_lc, o_ln, o_nr,
          counts_s, accmax_s, feat_s, attc_s, repc_s, pres_s, ksq_s,
          va_s, vr_s, nr_s, smem_s):
    p = pl.program_id(0)
    i = pl.program_id(1)
    f32 = jnp.float32

    @pl.when((p == 0) & (i == 0))
    def _init():
        counts_s[...] = jnp.zeros((1, _T), f32)
        accmax_s[...] = jnp.full((1, _T), -1.0, f32)
        feat_s[...] = jnp.zeros((16, _T), f32)
        smem_s[0] = 0.0
        smem_s[1] = 0.0

    oid = oid_ref[...]
    beta = beta_ref[...]
    ath = 0.5 * (jnp.log1p(beta) - jnp.log1p(-beta))
    q = ath * ath + _QMIN
    cols = lax.broadcasted_iota(jnp.int32, (_B, _T), 1)

    @pl.when(p == 0)
    def _phase0():
        m = oid == cols
        counts_s[...] += jnp.sum(m.astype(f32), axis=0, keepdims=True)
        qcol = jnp.where(m, q, -1.0)
        lmax = jnp.max(qcol, axis=0, keepdims=True)
        rows = lax.broadcasted_iota(jnp.int32, (_B, _T), 0)
        ismax = m & (qcol == lmax)
        larg = jnp.min(jnp.where(ismax, rows, _BIG), axis=0, keepdims=True)
        onehot = (rows == larg).astype(f32)
        feats = jnp.concatenate(
            [x_ref[...], q, beta, jnp.zeros((_B, 6), f32)], axis=1)
        cand = lax.dot_general(feats, onehot, (((0,), (0,)), ((), ())),
                               preferred_element_type=f32,
                               precision=lax.Precision.HIGHEST)
        upd = lmax > accmax_s[...]
        feat_s[...] = jnp.where(upd, cand, feat_s[...])
        accmax_s[...] = jnp.where(upd, lmax, accmax_s[...])
        nm = (oid == 0).astype(f32)
        smem_s[0] += jnp.sum(beta * nm)
        smem_s[1] += jnp.sum(nm)

    @pl.when((p == 0) & (i == _NB - 1))
    def _epilogue():
        counts = counts_s[...]
        tcols = lax.broadcasted_iota(jnp.int32, (1, _T), 1)
        pres = (counts > 0.0) & (tcols > 0)
        n_obj = jnp.sum(pres.astype(f32))
        q_k = feat_s[8:9, :]
        beta_k = feat_s[9:10, :]
        xkT = feat_s[0:8, :]
        ksq_s[...] = jnp.sum(xkT * xkT, axis=0, keepdims=True)
        attc_s[...] = jnp.where(pres, q_k / (counts * n_obj), 0.0)
        rep_norm = jnp.maximum((f32(_N) - counts) * n_obj, 1.0)
        repc_s[...] = jnp.where(pres, q_k / rep_norm, 0.0)
        pres_s[...] = pres.astype(f32)
        smem_s[2] = jnp.sum(jnp.where(pres, 1.0 - beta_k, 0.0)) / n_obj
        va_s[...] = jnp.zeros((1, _T), f32)
        vr_s[...] = jnp.zeros((1, _T), f32)
        nr_s[...] = jnp.zeros((1, _T), jnp.int32)

    @pl.when(p == 1)
    def _phase1():
        x = x_ref[...]
        xkT = feat_s[0:8, :]
        g = lax.dot_general(x, xkT, (((1,), (0,)), ((), ())),
                            preferred_element_type=f32,
                            precision=lax.Precision.HIGHEST)
        xsq = jnp.sum(x * x, axis=1, keepdims=True)
        d2 = jnp.maximum(xsq + ksq_s[...] - 2.0 * g, 0.0)
        dist = jnp.sqrt(jnp.maximum(d2, 1e-12))
        att = (oid == cols)
        va_s[...] += jnp.sum(
            jnp.where(att, (q * attc_s[...]) * d2, 0.0), axis=0, keepdims=True)
        rep = (pres_s[...] > 0.0) & (~att) & (dist < 1.0)
        vr_s[...] += jnp.sum(
            jnp.where(rep, (q * repc_s[...]) * (1.0 - dist), 0.0),
            axis=0, keepdims=True)
        nr_s[...] += jnp.sum(rep.astype(jnp.int32), axis=0, keepdims=True)

    @pl.when((p == 1) & (i == _NB - 1))
    def _final():
        va = jnp.sum(va_s[...])
        vr = jnp.sum(vr_s[...])
        nr = jnp.sum(nr_s[...]).astype(f32)
        lc = smem_s[2]
        ln = smem_s[0] / smem_s[1]
        loss = va + vr + lc + jnp.where(jnp.isnan(ln), 0.0, ln) * _SB
        o_loss[...] = loss.reshape(1, 1)
        o_va[...] = va.reshape(1, 1)
        o_vr[...] = vr.reshape(1, 1)
        o_lc[...] = jnp.full((1, 1), lc, f32)
        o_ln[...] = jnp.full((1, 1), ln, f32)
        o_nr[...] = nr.reshape(1, 1)


def kernel(hit_score, hit_embedding, hit_particle_id):
    beta = hit_score.reshape(_N, 1)
    oid = hit_particle_id.reshape(_N, 1).astype(jnp.int32)
    x = hit_embedding

    scalar = jax.ShapeDtypeStruct((1, 1), jnp.float32)
    outs = pl.pallas_call(
        _body,
        grid=(2, _NB),
        in_specs=[
            pl.BlockSpec((_B, _D), lambda p, i: (i, 0)),
            pl.BlockSpec((_B, 1), lambda p, i: (i, 0)),
            pl.BlockSpec((_B, 1), lambda p, i: (i, 0)),
        ],
        out_specs=[pl.BlockSpec((1, 1), lambda p, i: (0, 0))] * 6,
        out_shape=[scalar] * 6,
        scratch_shapes=[
            pltpu.VMEM((1, _T), jnp.float32),
            pltpu.VMEM((1, _T), jnp.float32),
            pltpu.VMEM((16, _T), jnp.float32),
            pltpu.VMEM((1, _T), jnp.float32),
            pltpu.VMEM((1, _T), jnp.float32),
            pltpu.VMEM((1, _T), jnp.float32),
            pltpu.VMEM((1, _T), jnp.float32),
            pltpu.VMEM((1, _T), jnp.float32),
            pltpu.VMEM((1, _T), jnp.float32),
            pltpu.VMEM((1, _T), jnp.int32),
            pltpu.SMEM((4,), jnp.float32),
        ],
        compiler_params=pltpu.CompilerParams(
            dimension_semantics=("arbitrary", "arbitrary")),
    )(x, beta, oid)

    loss, va, vr, lc, ln, nr = [o[0, 0] for o in outs]
    return (loss, va, vr, lc, ln, nr)
